# baseline (speedup 1.0000x reference)
import functools

import jax
import jax.numpy as jnp
from jax import lax
from jax.experimental import pallas as pl
from jax.experimental.pallas import tpu as pltpu
from jax.experimental.pallas import tpu_sc as plsc

_DIM = 2048
_NE = 64
_NT = 16384
_TILE = 1024

_NC, _NS, _L = 2, 16, 16
_NW = _NC * _NS
_TPW = _NT // _NW
_GROUPS = _TPW // _L


def _matmul_body(h2_ref, b_ref, out_ref):
    out_ref[...] = jnp.dot(h2_ref[...], b_ref[...],
                           preferred_element_type=jnp.float32)


def _gate_logits_chunk(hidden2, bdiag, start_blk, ntok):
    return pl.pallas_call(
        _matmul_body,
        grid=(ntok // _TILE,),
        in_specs=[
            pl.BlockSpec((_TILE // 2, 2 * _DIM), lambda i: (start_blk + i, 0)),
            pl.BlockSpec((2 * _DIM, 2 * _NE), lambda i: (0, 0)),
        ],
        out_specs=pl.BlockSpec((_TILE // 2, 2 * _NE), lambda i: (i, 0)),
        out_shape=jax.ShapeDtypeStruct((ntok // 2, 2 * _NE), jnp.float32),
    )(hidden2, bdiag)


def _topk_groups(lbuf, ibuf, sbuf, groups):
    lane = lax.iota(jnp.int32, _L)
    zero_i = jnp.zeros((_L,), jnp.int32)

    def group(g, carry):
        rbase = (g * _L + lane) * _NE
        m1 = plsc.load_gather(lbuf, [rbase])
        i1 = zero_i
        m2 = jnp.full((_L,), -jnp.inf, jnp.float32)
        i2 = zero_i
        for e in range(1, _NE):
            e_vec = jnp.full((_L,), e, jnp.int32)
            v = plsc.load_gather(lbuf, [rbase + e])
            gt1 = v > m1
            lo = jnp.where(gt1, m1, v)
            li = jnp.where(gt1, i1, e_vec)
            m1 = jnp.where(gt1, v, m1)
            i1 = jnp.where(gt1, e_vec, i1)
            gt2 = lo > m2
            m2 = jnp.where(gt2, lo, m2)
            i2 = jnp.where(gt2, li, i2)
        ex = jnp.exp(m2 - m1)
        s1 = 1.0 / (1.0 + ex)
        s2 = 1.0 - s1
        p0 = g * (2 * _L) + 2 * lane
        plsc.store_scatter(ibuf, [p0], i1)
        plsc.store_scatter(ibuf, [p0 + 1], i2)
        plsc.store_scatter(sbuf, [p0], s1)
        plsc.store_scatter(sbuf, [p0 + 1], s2)
        return carry

    lax.fori_loop(0, groups, group, 0)


def _make_topk_sc(ntok):
    tpw = ntok // _NW
    groups = tpw // _L

    def _topk_body(lg_hbm, idx_hbm, sc_hbm, lbuf, ibuf, sbuf):
        wid = lax.axis_index("s") * _NC + lax.axis_index("c")
        base = wid * tpw
        pltpu.sync_copy(lg_hbm.at[pl.ds(base * _NE, tpw * _NE)], lbuf)
        _topk_groups(lbuf, ibuf, sbuf, groups)
        pltpu.sync_copy(ibuf, idx_hbm.at[pl.ds(2 * base, 2 * tpw)])
        pltpu.sync_copy(sbuf, sc_hbm.at[pl.ds(2 * base, 2 * tpw)])

    return pl.kernel(
        _topk_body,
        out_type=[
            jax.ShapeDtypeStruct((2 * ntok,), jnp.int32),
            jax.ShapeDtypeStruct((2 * ntok,), jnp.float32),
        ],
        mesh=plsc.VectorSubcoreMesh(core_axis_name="c", subcore_axis_name="s"),
        compiler_params=pltpu.CompilerParams(needs_layout_passes=False),
        scratch_types=[
            pltpu.VMEM((tpw * _NE,), jnp.float32),
            pltpu.VMEM((2 * tpw,), jnp.int32),
            pltpu.VMEM((2 * tpw,), jnp.float32),
        ],
    )


_CHUNKS = (10240, 4096, 2048)
_TOPK_SC = {n: _make_topk_sc(n) for n in set(_CHUNKS)}


def kernel(hidden_states, gate_weight):
    hidden2 = hidden_states.reshape(_NT // 2, 2 * _DIM)
    wt = gate_weight.T
    bdiag = jnp.concatenate(
        [jnp.pad(wt, ((0, 0), (0, _NE))),
         jnp.pad(wt, ((0, 0), (_NE, 0)))], axis=0)
    idx_parts, sc_parts = [], []
    start = 0
    for ntok in _CHUNKS:
        logits = _gate_logits_chunk(hidden2, bdiag,
                                    start // _TILE, ntok)
        idx_flat, sc_flat = _TOPK_SC[ntok](logits.reshape(ntok * _NE))
        idx_parts.append(idx_flat)
        sc_parts.append(sc_flat)
        start += ntok
    return (jnp.concatenate(idx_parts).reshape(_NT, 2),
            jnp.concatenate(sc_parts).reshape(_NT, 2))


# speedup vs baseline: 2.2621x; 2.2621x over previous
import functools

import jax
import jax.numpy as jnp
from jax import lax
from jax.experimental import pallas as pl
from jax.experimental.pallas import tpu as pltpu
from jax.experimental.pallas import tpu_sc as plsc

_DIM = 2048
_NE = 64
_NT = 16384
_TILE = 1024

_NC, _NS, _L = 2, 16, 16
_NW = _NC * _NS
_TPW = _NT // _NW
_GROUPS = _TPW // _L


def _matmul_body(ha_ref, hb_ref, w_ref, out_ref):
    dims = (((1,), (1,)), ((), ()))
    da = lax.dot_general(ha_ref[...], w_ref[...], dims,
                         preferred_element_type=jnp.float32)
    db = lax.dot_general(hb_ref[...], w_ref[...], dims,
                         preferred_element_type=jnp.float32)
    out_ref[...] = jnp.concatenate([da, db], axis=1)


def _gate_logits_chunk(hidden_states, gate_weight, start_tok, ntok):
    s0 = start_tok // (_TILE // 2)
    half = ntok // _TILE
    return pl.pallas_call(
        _matmul_body,
        grid=(ntok // _TILE,),
        in_specs=[
            pl.BlockSpec((_TILE // 2, _DIM), lambda i: (s0 + i, 0)),
            pl.BlockSpec((_TILE // 2, _DIM), lambda i: (s0 + half + i, 0)),
            pl.BlockSpec((_NE, _DIM), lambda i: (0, 0)),
        ],
        out_specs=pl.BlockSpec((_TILE // 2, 2 * _NE), lambda i: (i, 0)),
        out_shape=jax.ShapeDtypeStruct((ntok // 2, 2 * _NE), jnp.float32),
    )(hidden_states, hidden_states, gate_weight)


def _scan16(lbuf, fbase, ibuf, sbuf, pbase):
    zero_i = jnp.zeros((_L,), jnp.int32)
    m1 = plsc.load_gather(lbuf, [fbase])
    i1 = zero_i
    m2 = jnp.full((_L,), -jnp.inf, jnp.float32)
    i2 = zero_i
    for e in range(1, _NE):
        e_vec = jnp.full((_L,), e, jnp.int32)
        v = plsc.load_gather(lbuf, [fbase + e])
        gt1 = v > m1
        lo = jnp.where(gt1, m1, v)
        li = jnp.where(gt1, i1, e_vec)
        m1 = jnp.where(gt1, v, m1)
        i1 = jnp.where(gt1, e_vec, i1)
        gt2 = lo > m2
        m2 = jnp.where(gt2, lo, m2)
        i2 = jnp.where(gt2, li, i2)
    ex = jnp.exp(m2 - m1)
    s1 = 1.0 / (1.0 + ex)
    s2 = 1.0 - s1
    plsc.store_scatter(ibuf, [pbase], i1)
    plsc.store_scatter(ibuf, [pbase + 1], i2)
    plsc.store_scatter(sbuf, [pbase], s1)
    plsc.store_scatter(sbuf, [pbase + 1], s2)


def _make_topk_sc(ntok):
    rpw = ntok // 2 // _NW
    groups = rpw // _L

    def _topk_body(lg_hbm, idx_hbm, sc_hbm, lbuf, ibuf, sbuf):
        wid = lax.axis_index("s") * _NC + lax.axis_index("c")
        rbase = wid * rpw
        pltpu.sync_copy(lg_hbm.at[pl.ds(rbase * 2 * _NE, rpw * 2 * _NE)], lbuf)

        lane = lax.iota(jnp.int32, _L)

        def group(g, carry):
            rows = g * _L + lane
            fb = rows * (2 * _NE)
            _scan16(lbuf, fb, ibuf, sbuf, 2 * rows)
            _scan16(lbuf, fb + _NE, ibuf, sbuf, 2 * rpw + 2 * rows)
            return carry

        lax.fori_loop(0, groups, group, 0)
        pltpu.sync_copy(ibuf.at[pl.ds(0, 2 * rpw)],
                        idx_hbm.at[pl.ds(2 * rbase, 2 * rpw)])
        pltpu.sync_copy(ibuf.at[pl.ds(2 * rpw, 2 * rpw)],
                        idx_hbm.at[pl.ds(ntok + 2 * rbase, 2 * rpw)])
        pltpu.sync_copy(sbuf.at[pl.ds(0, 2 * rpw)],
                        sc_hbm.at[pl.ds(2 * rbase, 2 * rpw)])
        pltpu.sync_copy(sbuf.at[pl.ds(2 * rpw, 2 * rpw)],
                        sc_hbm.at[pl.ds(ntok + 2 * rbase, 2 * rpw)])

    return pl.kernel(
        _topk_body,
        out_type=[
            jax.ShapeDtypeStruct((2 * ntok,), jnp.int32),
            jax.ShapeDtypeStruct((2 * ntok,), jnp.float32),
        ],
        mesh=plsc.VectorSubcoreMesh(core_axis_name="c", subcore_axis_name="s"),
        compiler_params=pltpu.CompilerParams(needs_layout_passes=False),
        scratch_types=[
            pltpu.VMEM((rpw * 2 * _NE,), jnp.float32),
            pltpu.VMEM((4 * rpw,), jnp.int32),
            pltpu.VMEM((4 * rpw,), jnp.float32),
        ],
    )


_CHUNKS = (10240, 4096, 2048)
_TOPK_SC = {n: _make_topk_sc(n) for n in set(_CHUNKS)}


def kernel(hidden_states, gate_weight):
    idx_parts, sc_parts = [], []
    start = 0
    for ntok in _CHUNKS:
        logits = _gate_logits_chunk(hidden_states, gate_weight,
                                    start, ntok)
        idx_flat, sc_flat = _TOPK_SC[ntok](logits.reshape(ntok * _NE))
        idx_parts.append(idx_flat)
        sc_parts.append(sc_flat)
        start += ntok
    return (jnp.concatenate(idx_parts).reshape(_NT, 2),
            jnp.concatenate(sc_parts).reshape(_NT, 2))
